# Initial kernel scaffold; baseline (speedup 1.0000x reference)
import jax
import jax.numpy as jnp
from jax.experimental import pallas as pl


def kernel(x, edge_index, W1, b1, Wx, bx, Wfc, bfc):
    raise NotImplementedError("write your pallas kernel here")


# speedup vs baseline: 8.7984x; 8.7984x over previous
import functools

import jax
import jax.numpy as jnp
from jax import lax
from jax.experimental import pallas as pl
from jax.experimental.pallas import tpu as pltpu
from jax.experimental.pallas import tpu_sc as plsc

N = 10000
E = 320000
F = 128
NCLASS = 40

NP = 10240
NSUB = 16
NCORE = 2
NW = NCORE * NSUB
C = 128
K = 80
EPAD = NW * K * C
ROWS_PER_SUB = NP // NSUB

_mesh = plsc.VectorSubcoreMesh(core_axis_name="c", subcore_axis_name="s")


@functools.partial(
    pl.kernel,
    mesh=_mesh,
    out_type=jax.ShapeDtypeStruct((NCORE, NP, F), jnp.float32),
    scratch_types=[
        pltpu.VMEM((K, C), jnp.int32),
        pltpu.VMEM((C, F), jnp.float32),
        pltpu.VMEM_SHARED((NP, F), jnp.float32),
    ],
)
def _deg(dst_hbm, ones_hbm, zeros_hbm, out_hbm, dst_v, ones_v, acc):
    cid = lax.axis_index("c")
    sid = lax.axis_index("s")
    w = cid * NSUB + sid
    r0 = sid * ROWS_PER_SUB
    pltpu.sync_copy(zeros_hbm, acc.at[pl.ds(r0, ROWS_PER_SUB)])
    pltpu.sync_copy(ones_hbm, ones_v)
    pltpu.sync_copy(dst_hbm.at[pl.ds(w * K, K)], dst_v)
    plsc.subcore_barrier()

    def body(j, carry):
        pltpu.sync_copy(ones_v, acc.at[dst_v.at[j]], add=True)
        return carry

    lax.fori_loop(0, K, body, 0)
    plsc.subcore_barrier()
    pltpu.sync_copy(acc.at[pl.ds(r0, ROWS_PER_SUB)],
                    out_hbm.at[cid, pl.ds(r0, ROWS_PER_SUB)])


@functools.partial(
    pl.kernel,
    mesh=_mesh,
    out_type=jax.ShapeDtypeStruct((NCORE, NP, F), jnp.float32),
    scratch_types=[
        pltpu.VMEM((K, C), jnp.int32),
        pltpu.VMEM((K, C), jnp.int32),
        pltpu.VMEM((C, F), jnp.float32),
        pltpu.VMEM_SHARED((NP, F), jnp.float32),
        pltpu.SemaphoreType.DMA,
    ],
)
def _agg(hs_hbm, src_hbm, dst_hbm, zeros_hbm, out_hbm,
         src_v, dst_v, rows_v, acc, sem):
    cid = lax.axis_index("c")
    sid = lax.axis_index("s")
    w = cid * NSUB + sid
    r0 = sid * ROWS_PER_SUB
    pltpu.sync_copy(zeros_hbm, acc.at[pl.ds(r0, ROWS_PER_SUB)])
    pltpu.sync_copy(src_hbm.at[pl.ds(w * K, K)], src_v)
    pltpu.sync_copy(dst_hbm.at[pl.ds(w * K, K)], dst_v)
    plsc.subcore_barrier()

    def body(j, carry):
        pltpu.async_copy(hs_hbm.at[src_v.at[j]], rows_v, sem).wait()
        pltpu.sync_copy(rows_v, acc.at[dst_v.at[j]], add=True)
        return carry

    lax.fori_loop(0, K, body, 0)
    plsc.subcore_barrier()
    pltpu.sync_copy(acc.at[pl.ds(r0, ROWS_PER_SUB)],
                    out_hbm.at[cid, pl.ds(r0, ROWS_PER_SUB)])


_BLK = 512
_GRID = NP // _BLK


def _dis_from(deg_ref):
    deg = deg_ref[0, :, 0:1] + deg_ref[1, :, 0:1] + 1.0
    return lax.rsqrt(deg)


def _tc1_body(deg_ref, x_ref, w_ref, hs_ref):
    dis = _dis_from(deg_ref)
    h = jnp.dot(x_ref[...], w_ref[...], preferred_element_type=jnp.float32)
    hs_ref[...] = h * dis


def _tc2_body(p_ref, hs1_ref, deg_ref, b_ref, w_ref, x1_ref, hs2_ref):
    dis = _dis_from(deg_ref)
    agg = p_ref[0] + p_ref[1] + hs1_ref[...]
    x1 = jnp.maximum(agg * dis + b_ref[...], 0.0)
    x1_ref[...] = x1
    h2 = jnp.dot(x1, w_ref[...], preferred_element_type=jnp.float32)
    hs2_ref[...] = h2 * dis


def _tc3_body(p_ref, hs2_ref, deg_ref, b_ref, x1_ref, wfc_ref, bfc_ref, o_ref):
    dis = _dis_from(deg_ref)
    agg = p_ref[0] + p_ref[1] + hs2_ref[...]
    x2 = jnp.maximum(agg * dis + b_ref[...], 0.0)
    h = jnp.maximum(x1_ref[...], x2)
    o_ref[...] = jnp.dot(h, wfc_ref[...],
                         preferred_element_type=jnp.float32) + bfc_ref[...]


def _row_spec(shape_cols):
    return pl.BlockSpec((_BLK, shape_cols), lambda i: (i, 0))


_deg_spec = pl.BlockSpec((2, _BLK, F), lambda i: (0, i, 0))
_p_spec = pl.BlockSpec((2, _BLK, F), lambda i: (0, i, 0))
_w_spec = pl.BlockSpec((F, F), lambda i: (0, 0))
_b_spec = pl.BlockSpec((1, F), lambda i: (0, 0))

_tc1 = pl.pallas_call(
    _tc1_body,
    grid=(_GRID,),
    in_specs=[_deg_spec, _row_spec(F), _w_spec],
    out_specs=_row_spec(F),
    out_shape=jax.ShapeDtypeStruct((NP, F), jnp.float32),
)

_tc2 = pl.pallas_call(
    _tc2_body,
    grid=(_GRID,),
    in_specs=[_p_spec, _row_spec(F), _deg_spec, _b_spec, _w_spec],
    out_specs=[_row_spec(F), _row_spec(F)],
    out_shape=[jax.ShapeDtypeStruct((NP, F), jnp.float32),
               jax.ShapeDtypeStruct((NP, F), jnp.float32)],
)

_tc3 = pl.pallas_call(
    _tc3_body,
    grid=(_GRID,),
    in_specs=[_p_spec, _row_spec(F), _deg_spec, _b_spec, _row_spec(F),
              _w_spec, _b_spec],
    out_specs=_row_spec(F),
    out_shape=jax.ShapeDtypeStruct((NP, F), jnp.float32),
)


def kernel(x, edge_index, W1, b1, Wx, bx, Wfc, bfc):
    ei = edge_index.astype(jnp.int32)
    pad = EPAD - E
    src = jnp.concatenate([ei[0], jnp.zeros((pad,), jnp.int32)])
    dst = jnp.concatenate([ei[1], jnp.full((pad,), N, jnp.int32)])
    src = src.reshape(NW * K, C)
    dst = dst.reshape(NW * K, C)

    x_p = jnp.pad(x, ((0, NP - N), (0, 0)))
    onesF = jnp.ones((C, F), jnp.float32)
    zF = jnp.zeros((ROWS_PER_SUB, F), jnp.float32)
    b1_r = b1.reshape(1, F)
    bx_r = bx.reshape(1, F)
    Wfc_p = jnp.pad(Wfc, ((0, 0), (0, F - NCLASS)))
    bfc_p = jnp.pad(bfc, ((0, F - NCLASS),)).reshape(1, F)

    degp = _deg(dst, onesF, zF)
    hs1 = _tc1(degp, x_p, W1)
    p1 = _agg(hs1, src, dst, zF)
    x1, hs2 = _tc2(p1, hs1, degp, b1_r, Wx)
    p2 = _agg(hs2, src, dst, zF)
    out = _tc3(p2, hs2, degp, bx_r, x1, Wfc_p, bfc_p)
    return out[:N, :NCLASS]


# speedup vs baseline: 9.8743x; 1.1223x over previous
import functools

import jax
import jax.numpy as jnp
from jax import lax
from jax.experimental import pallas as pl
from jax.experimental.pallas import tpu as pltpu
from jax.experimental.pallas import tpu_sc as plsc

N = 10000
E = 320000
F = 128
NCLASS = 40

NP = 10240
NSUB = 16
NCORE = 2
NW = NCORE * NSUB
C = 128
K = 80
EPAD = NW * K * C
ROWS_PER_SUB = NP // NSUB

_mesh = plsc.VectorSubcoreMesh(core_axis_name="c", subcore_axis_name="s")


@functools.partial(
    pl.kernel,
    mesh=_mesh,
    out_type=jax.ShapeDtypeStruct((NCORE, NP, F), jnp.float32),
    scratch_types=[
        pltpu.VMEM((K, C), jnp.int32),
        pltpu.VMEM((C, F), jnp.float32),
        pltpu.VMEM_SHARED((NP, F), jnp.float32),
    ],
)
def _deg(dst_hbm, ones_hbm, zeros_hbm, out_hbm, dst_v, ones_v, acc):
    cid = lax.axis_index("c")
    sid = lax.axis_index("s")
    w = cid * NSUB + sid
    r0 = sid * ROWS_PER_SUB
    pltpu.sync_copy(zeros_hbm, acc.at[pl.ds(r0, ROWS_PER_SUB)])
    pltpu.sync_copy(ones_hbm, ones_v)
    pltpu.sync_copy(dst_hbm.at[pl.ds(w * K, K)], dst_v)
    plsc.subcore_barrier()

    def body(j, carry):
        pltpu.sync_copy(ones_v, acc.at[dst_v.at[j]], add=True)
        return carry

    lax.fori_loop(0, K, body, 0)
    plsc.subcore_barrier()
    pltpu.sync_copy(acc.at[pl.ds(r0, ROWS_PER_SUB)],
                    out_hbm.at[cid, pl.ds(r0, ROWS_PER_SUB)])


NBUF = 2
K2 = K // 2


@functools.partial(
    pl.kernel,
    mesh=_mesh,
    out_type=jax.ShapeDtypeStruct((NCORE, NP, F), jnp.float32),
    scratch_types=[
        pltpu.VMEM((K2, C), jnp.int32),
        pltpu.VMEM((K2, C), jnp.int32),
        pltpu.VMEM((NBUF, C, F), jnp.float32),
        pltpu.VMEM_SHARED((NP, F), jnp.float32),
        pltpu.SemaphoreType.DMA,
        pltpu.SemaphoreType.DMA,
    ],
)
def _agg(hs_hbm, src_hbm, dst_hbm, zeros_hbm, out_hbm,
         src_v, dst_v, rows_v, acc, s0, s1):
    cid = lax.axis_index("c")
    sid = lax.axis_index("s")
    w = cid * NSUB + sid
    r0 = sid * ROWS_PER_SUB
    pltpu.sync_copy(zeros_hbm, acc.at[pl.ds(r0, ROWS_PER_SUB)])
    plsc.subcore_barrier()

    sems = (s0, s1)
    for h in range(2):
        pltpu.sync_copy(src_hbm.at[pl.ds(w * K + h * K2, K2)], src_v)
        pltpu.sync_copy(dst_hbm.at[pl.ds(w * K + h * K2, K2)], dst_v)
        for b in range(NBUF):
            pltpu.async_copy(hs_hbm.at[src_v.at[b]], rows_v.at[b], sems[b])

        def outer(i, carry):
            j0 = i * NBUF
            for b in range(NBUF):
                j = j0 + b
                pltpu.make_async_copy(hs_hbm.at[pl.ds(0, C)], rows_v.at[b],
                                      sems[b]).wait()
                pltpu.sync_copy(rows_v.at[b], acc.at[dst_v.at[j]], add=True)

                @pl.when(j + NBUF < K2)
                def _():
                    pltpu.async_copy(hs_hbm.at[src_v.at[j + NBUF]],
                                     rows_v.at[b], sems[b])
            return carry

        lax.fori_loop(0, K2 // NBUF, outer, 0)

    plsc.subcore_barrier()
    pltpu.sync_copy(acc.at[pl.ds(r0, ROWS_PER_SUB)],
                    out_hbm.at[cid, pl.ds(r0, ROWS_PER_SUB)])


_BLK = 512
_GRID = NP // _BLK


def _dis_from(deg_ref):
    deg = deg_ref[0, :, 0:1] + deg_ref[1, :, 0:1] + 1.0
    return lax.rsqrt(deg)


def _tc1_body(deg_ref, x_ref, w_ref, hs_ref):
    dis = _dis_from(deg_ref)
    h = jnp.dot(x_ref[...], w_ref[...], preferred_element_type=jnp.float32)
    hs_ref[...] = h * dis


def _tc2_body(p_ref, hs1_ref, deg_ref, b_ref, w_ref, x1_ref, hs2_ref):
    dis = _dis_from(deg_ref)
    agg = p_ref[0] + p_ref[1] + hs1_ref[...]
    x1 = jnp.maximum(agg * dis + b_ref[...], 0.0)
    x1_ref[...] = x1
    h2 = jnp.dot(x1, w_ref[...], preferred_element_type=jnp.float32)
    hs2_ref[...] = h2 * dis


def _tc3_body(p_ref, hs2_ref, deg_ref, b_ref, x1_ref, wfc_ref, bfc_ref, o_ref):
    dis = _dis_from(deg_ref)
    agg = p_ref[0] + p_ref[1] + hs2_ref[...]
    x2 = jnp.maximum(agg * dis + b_ref[...], 0.0)
    h = jnp.maximum(x1_ref[...], x2)
    o_ref[...] = jnp.dot(h, wfc_ref[...],
                         preferred_element_type=jnp.float32) + bfc_ref[...]


def _row_spec(shape_cols):
    return pl.BlockSpec((_BLK, shape_cols), lambda i: (i, 0))


_deg_spec = pl.BlockSpec((2, _BLK, F), lambda i: (0, i, 0))
_p_spec = pl.BlockSpec((2, _BLK, F), lambda i: (0, i, 0))
_w_spec = pl.BlockSpec((F, F), lambda i: (0, 0))
_b_spec = pl.BlockSpec((1, F), lambda i: (0, 0))

_tc1 = pl.pallas_call(
    _tc1_body,
    grid=(_GRID,),
    in_specs=[_deg_spec, _row_spec(F), _w_spec],
    out_specs=_row_spec(F),
    out_shape=jax.ShapeDtypeStruct((NP, F), jnp.float32),
)

_tc2 = pl.pallas_call(
    _tc2_body,
    grid=(_GRID,),
    in_specs=[_p_spec, _row_spec(F), _deg_spec, _b_spec, _w_spec],
    out_specs=[_row_spec(F), _row_spec(F)],
    out_shape=[jax.ShapeDtypeStruct((NP, F), jnp.float32),
               jax.ShapeDtypeStruct((NP, F), jnp.float32)],
)

_tc3 = pl.pallas_call(
    _tc3_body,
    grid=(_GRID,),
    in_specs=[_p_spec, _row_spec(F), _deg_spec, _b_spec, _row_spec(F),
              _w_spec, _b_spec],
    out_specs=_row_spec(F),
    out_shape=jax.ShapeDtypeStruct((NP, F), jnp.float32),
)


def kernel(x, edge_index, W1, b1, Wx, bx, Wfc, bfc):
    ei = edge_index.astype(jnp.int32)
    pad = EPAD - E
    src = jnp.concatenate([ei[0], jnp.zeros((pad,), jnp.int32)])
    dst = jnp.concatenate([ei[1], jnp.full((pad,), N, jnp.int32)])
    src = src.reshape(NW * K, C)
    dst = dst.reshape(NW * K, C)

    x_p = jnp.pad(x, ((0, NP - N), (0, 0)))
    onesF = jnp.ones((C, F), jnp.float32)
    zF = jnp.zeros((ROWS_PER_SUB, F), jnp.float32)
    b1_r = b1.reshape(1, F)
    bx_r = bx.reshape(1, F)
    Wfc_p = jnp.pad(Wfc, ((0, 0), (0, F - NCLASS)))
    bfc_p = jnp.pad(bfc, ((0, F - NCLASS),)).reshape(1, F)

    degp = _deg(dst, onesF, zF)
    hs1 = _tc1(degp, x_p, W1)
    p1 = _agg(hs1, src, dst, zF)
    x1, hs2 = _tc2(p1, hs1, degp, b1_r, Wx)
    p2 = _agg(hs2, src, dst, zF)
    out = _tc3(p2, hs2, degp, bx_r, x1, Wfc_p, bfc_p)
    return out[:N, :NCLASS]


# speedup vs baseline: 10.3389x; 1.0471x over previous
import functools

import jax
import jax.numpy as jnp
from jax import lax
from jax.experimental import pallas as pl
from jax.experimental.pallas import tpu as pltpu
from jax.experimental.pallas import tpu_sc as plsc

N = 10000
E = 320000
F = 128
NCLASS = 40

NP = 10240
NSUB = 16
NCORE = 2
NW = NCORE * NSUB
C = 128
K = 80
EPAD = NW * K * C
ROWS_PER_SUB = NP // NSUB

_mesh = plsc.VectorSubcoreMesh(core_axis_name="c", subcore_axis_name="s")


@functools.partial(
    pl.kernel,
    mesh=_mesh,
    out_type=jax.ShapeDtypeStruct((NCORE, NP, F), jnp.float32),
    scratch_types=[
        pltpu.VMEM((K, C), jnp.int32),
        pltpu.VMEM((C, F), jnp.float32),
        pltpu.VMEM_SHARED((NP, F), jnp.float32),
    ],
)
def _deg(dst_hbm, ones_hbm, zeros_hbm, out_hbm, dst_v, ones_v, acc):
    cid = lax.axis_index("c")
    sid = lax.axis_index("s")
    w = cid * NSUB + sid
    r0 = sid * ROWS_PER_SUB
    pltpu.sync_copy(zeros_hbm, acc.at[pl.ds(r0, ROWS_PER_SUB)])
    pltpu.sync_copy(ones_hbm, ones_v)
    pltpu.sync_copy(dst_hbm.at[pl.ds(w * K, K)], dst_v)
    plsc.subcore_barrier()

    def body(j, carry):
        pltpu.sync_copy(ones_v, acc.at[dst_v.at[j]], add=True)
        return carry

    lax.fori_loop(0, K, body, 0)
    plsc.subcore_barrier()
    pltpu.sync_copy(acc.at[pl.ds(r0, ROWS_PER_SUB)],
                    out_hbm.at[cid, pl.ds(r0, ROWS_PER_SUB)])


NBUF = 2
K2 = 40
K0 = 120
K1 = 40


@functools.partial(
    pl.kernel,
    mesh=_mesh,
    out_type=jax.ShapeDtypeStruct((NCORE, NP, F), jnp.float32),
    scratch_types=[
        pltpu.VMEM((K2, C), jnp.int32),
        pltpu.VMEM((K2, C), jnp.int32),
        pltpu.VMEM((NBUF, C, F), jnp.float32),
        pltpu.VMEM_SHARED((NP, F), jnp.float32),
        pltpu.SemaphoreType.DMA,
        pltpu.SemaphoreType.DMA,
    ],
)
def _agg(hs_hbm, src_hbm, dst_hbm, zeros_hbm, out_hbm,
         src_v, dst_v, rows_v, acc, s0, s1):
    cid = lax.axis_index("c")
    sid = lax.axis_index("s")
    r0 = sid * ROWS_PER_SUB
    pltpu.sync_copy(zeros_hbm, acc.at[pl.ds(r0, ROWS_PER_SUB)])
    plsc.subcore_barrier()

    sems = (s0, s1)

    def stage(base):
        pltpu.sync_copy(src_hbm.at[pl.ds(base, K2)], src_v)
        pltpu.sync_copy(dst_hbm.at[pl.ds(base, K2)], dst_v)
        for b in range(NBUF):
            pltpu.async_copy(hs_hbm.at[src_v.at[b]], rows_v.at[b], sems[b])

        def outer(i, carry):
            j0 = i * NBUF
            for b in range(NBUF):
                j = j0 + b
                pltpu.make_async_copy(hs_hbm.at[pl.ds(0, C)], rows_v.at[b],
                                      sems[b]).wait()
                pltpu.sync_copy(rows_v.at[b], acc.at[dst_v.at[j]], add=True)

                @pl.when(j + NBUF < K2)
                def _():
                    pltpu.async_copy(hs_hbm.at[src_v.at[j + NBUF]],
                                     rows_v.at[b], sems[b])
            return carry

        lax.fori_loop(0, K2 // NBUF, outer, 0)

    if K0 > 0:
        @pl.when(cid == 0)
        def _():
            for h in range(K0 // K2):
                stage(sid * K0 + h * K2)

    if K1 > 0:
        @pl.when(cid == 1)
        def _():
            for h in range(K1 // K2):
                stage(NSUB * K0 + sid * K1 + h * K2)

    plsc.subcore_barrier()
    pltpu.sync_copy(acc.at[pl.ds(r0, ROWS_PER_SUB)],
                    out_hbm.at[cid, pl.ds(r0, ROWS_PER_SUB)])


_BLK = 512
_GRID = NP // _BLK


def _dis_from(deg_ref):
    deg = deg_ref[0, :, 0:1] + deg_ref[1, :, 0:1] + 1.0
    return lax.rsqrt(deg)


def _tc1_body(deg_ref, x_ref, w_ref, hs_ref):
    dis = _dis_from(deg_ref)
    h = jnp.dot(x_ref[...], w_ref[...], preferred_element_type=jnp.float32)
    hs_ref[...] = h * dis


def _tc2_body(p_ref, hs1_ref, deg_ref, b_ref, w_ref, x1_ref, hs2_ref):
    dis = _dis_from(deg_ref)
    agg = p_ref[0] + p_ref[1] + hs1_ref[...]
    x1 = jnp.maximum(agg * dis + b_ref[...], 0.0)
    x1_ref[...] = x1
    h2 = jnp.dot(x1, w_ref[...], preferred_element_type=jnp.float32)
    hs2_ref[...] = h2 * dis


def _tc3_body(p_ref, hs2_ref, deg_ref, b_ref, x1_ref, wfc_ref, bfc_ref, o_ref):
    dis = _dis_from(deg_ref)
    agg = p_ref[0] + p_ref[1] + hs2_ref[...]
    x2 = jnp.maximum(agg * dis + b_ref[...], 0.0)
    h = jnp.maximum(x1_ref[...], x2)
    o_ref[...] = jnp.dot(h, wfc_ref[...],
                         preferred_element_type=jnp.float32) + bfc_ref[...]


def _row_spec(shape_cols):
    return pl.BlockSpec((_BLK, shape_cols), lambda i: (i, 0))


_deg_spec = pl.BlockSpec((2, _BLK, F), lambda i: (0, i, 0))
_p_spec = pl.BlockSpec((2, _BLK, F), lambda i: (0, i, 0))
_w_spec = pl.BlockSpec((F, F), lambda i: (0, 0))
_b_spec = pl.BlockSpec((1, F), lambda i: (0, 0))

_tc1 = pl.pallas_call(
    _tc1_body,
    grid=(_GRID,),
    in_specs=[_deg_spec, _row_spec(F), _w_spec],
    out_specs=_row_spec(F),
    out_shape=jax.ShapeDtypeStruct((NP, F), jnp.float32),
)

_tc2 = pl.pallas_call(
    _tc2_body,
    grid=(_GRID,),
    in_specs=[_p_spec, _row_spec(F), _deg_spec, _b_spec, _w_spec],
    out_specs=[_row_spec(F), _row_spec(F)],
    out_shape=[jax.ShapeDtypeStruct((NP, F), jnp.float32),
               jax.ShapeDtypeStruct((NP, F), jnp.float32)],
)

_tc3 = pl.pallas_call(
    _tc3_body,
    grid=(_GRID,),
    in_specs=[_p_spec, _row_spec(F), _deg_spec, _b_spec, _row_spec(F),
              _w_spec, _b_spec],
    out_specs=_row_spec(F),
    out_shape=jax.ShapeDtypeStruct((NP, F), jnp.float32),
)


def kernel(x, edge_index, W1, b1, Wx, bx, Wfc, bfc):
    ei = edge_index.astype(jnp.int32)
    pad = EPAD - E
    src = jnp.concatenate([ei[0], jnp.zeros((pad,), jnp.int32)])
    dst = jnp.concatenate([ei[1], jnp.full((pad,), N, jnp.int32)])
    src = src.reshape(NW * K, C)
    dst = dst.reshape(NW * K, C)

    x_p = jnp.pad(x, ((0, NP - N), (0, 0)))
    onesF = jnp.ones((C, F), jnp.float32)
    zF = jnp.zeros((ROWS_PER_SUB, F), jnp.float32)
    b1_r = b1.reshape(1, F)
    bx_r = bx.reshape(1, F)
    Wfc_p = jnp.pad(Wfc, ((0, 0), (0, F - NCLASS)))
    bfc_p = jnp.pad(bfc, ((0, F - NCLASS),)).reshape(1, F)

    degp = _deg(dst, onesF, zF)
    hs1 = _tc1(degp, x_p, W1)
    p1 = _agg(hs1, src, dst, zF)
    x1, hs2 = _tc2(p1, hs1, degp, b1_r, Wx)
    p2 = _agg(hs2, src, dst, zF)
    out = _tc3(p2, hs2, degp, bx_r, x1, Wfc_p, bfc_p)
    return out[:N, :NCLASS]
